# speedup vs baseline: 3.0476x; 3.0476x over previous
import functools

import jax
import jax.numpy as jnp
from jax import lax
from jax.experimental import pallas as pl
from jax.experimental.pallas import tpu as pltpu
from jax.experimental.pallas import tpu_sc as plsc

DIM = 128
CHUNK = 128
NC, NS = 2, 16
NW = NC * NS


@functools.lru_cache(maxsize=None)
def _gather_fn(n_per_w: int):
    mesh = plsc.VectorSubcoreMesh(core_axis_name="c", subcore_axis_name="s")

    @functools.partial(
        pl.kernel,
        mesh=mesh,
        out_type=jax.ShapeDtypeStruct((NW * n_per_w * CHUNK, DIM), jnp.float32),
        scratch_types=[
            pltpu.VMEM((n_per_w, CHUNK), jnp.int32),
            pltpu.VMEM((CHUNK, DIM), jnp.float32),
            pltpu.SemaphoreType.DMA,
        ],
    )
    def k(idx_hbm, table_hbm, out_hbm, idx_v, rows_v, sem):
        wid = lax.axis_index("s") * NC + lax.axis_index("c")
        cbase = wid * n_per_w
        pltpu.sync_copy(idx_hbm.at[pl.ds(cbase, n_per_w)], idx_v)

        def body(j, carry):
            pltpu.async_copy(table_hbm.at[idx_v.at[j]], rows_v, sem).wait()
            pltpu.sync_copy(rows_v, out_hbm.at[pl.ds((cbase + j) * CHUNK, CHUNK)])
            return carry

        lax.fori_loop(0, n_per_w, body, 0)

    return k


def kernel(input, table):
    flat = input.reshape(-1).astype(jnp.int32)
    n_chunks = flat.shape[0] // CHUNK
    idx2d = flat.reshape(n_chunks, CHUNK)
    out = _gather_fn(n_chunks // NW)(idx2d, table)
    return out.reshape(input.shape + (DIM,))


# speedup vs baseline: 3.4613x; 1.1357x over previous
import functools

import jax
import jax.numpy as jnp
from jax import lax
from jax.experimental import pallas as pl
from jax.experimental.pallas import tpu as pltpu
from jax.experimental.pallas import tpu_sc as plsc

DIM = 128
CHUNK = 128
NC, NS = 2, 16
NW = NC * NS


@functools.lru_cache(maxsize=None)
def _gather_fn(n_per_w: int):
    mesh = plsc.VectorSubcoreMesh(core_axis_name="c", subcore_axis_name="s")

    @functools.partial(
        pl.kernel,
        mesh=mesh,
        out_type=jax.ShapeDtypeStruct((NW * n_per_w * CHUNK, DIM), jnp.float32),
        scratch_types=[
            pltpu.VMEM((n_per_w, CHUNK), jnp.int32),
            pltpu.VMEM((CHUNK, DIM), jnp.float32),
            pltpu.VMEM((CHUNK, DIM), jnp.float32),
            pltpu.SemaphoreType.DMA,
            pltpu.SemaphoreType.DMA,
        ],
    )
    def k(idx_hbm, table_hbm, out_hbm, idx_v, rows0, rows1, gsem0, gsem1):
        wid = lax.axis_index("s") * NC + lax.axis_index("c")
        cbase = wid * n_per_w
        pltpu.sync_copy(idx_hbm.at[pl.ds(cbase, n_per_w)], idx_v)

        n2 = n_per_w // 2
        pltpu.async_copy(table_hbm.at[idx_v.at[0]], rows0, gsem0)

        def body(g, carry):
            j0 = g * 2
            j1 = j0 + 1
            pltpu.async_copy(table_hbm.at[idx_v.at[j1]], rows1, gsem1)
            pltpu.make_async_copy(table_hbm.at[idx_v.at[j0]], rows0, gsem0).wait()
            pltpu.sync_copy(rows0, out_hbm.at[pl.ds((cbase + j0) * CHUNK, CHUNK)])

            @pl.when(g + 1 < n2)
            def _():
                pltpu.async_copy(table_hbm.at[idx_v.at[j0 + 2]], rows0, gsem0)

            pltpu.make_async_copy(table_hbm.at[idx_v.at[j1]], rows1, gsem1).wait()
            pltpu.sync_copy(rows1, out_hbm.at[pl.ds((cbase + j1) * CHUNK, CHUNK)])
            return carry

        lax.fori_loop(0, n2, body, 0)

    return k


def kernel(input, table):
    flat = input.reshape(-1).astype(jnp.int32)
    n_chunks = flat.shape[0] // CHUNK
    idx2d = flat.reshape(n_chunks, CHUNK)
    out = _gather_fn(n_chunks // NW)(idx2d, table)
    return out.reshape(input.shape + (DIM,))
